# baseline (speedup 1.0000x reference)
import functools

import numpy as np
import jax
import jax.numpy as jnp
from jax import lax
from jax.experimental import pallas as pl
from jax.experimental.pallas import tpu as pltpu
from jax.experimental.pallas import tpu_sc as plsc

ROW_BLK = 512
CH = 32
LANES = 16


def _column_perm(D):
    w = np.arange(D // 2)
    g, j = w // LANES, w % LANES
    lo_cols = 2 * LANES * g + j
    return np.concatenate([lo_cols, lo_cols + LANES])


def _mm_kernel(t_ref, w_ref, b_ref, o_ref):
    y = (lax.dot_general(
        t_ref[...].astype(jnp.bfloat16), w_ref[...].astype(jnp.bfloat16),
        (((1,), (1,)), ((), ())),
        preferred_element_type=jnp.float32) + b_ref[...]).astype(jnp.bfloat16)
    h = y.shape[1] // 2
    lo = lax.bitcast_convert_type(y[:, :h], jnp.uint16).astype(jnp.uint32)
    hi = lax.bitcast_convert_type(y[:, h:], jnp.uint16).astype(jnp.uint32)
    o_ref[...] = lax.bitcast_convert_type(lo | (hi << 16), jnp.int32)


def _transform_table(table, W, b):
    V, D = table.shape
    grid = (pl.cdiv(V, ROW_BLK),)
    return pl.pallas_call(
        _mm_kernel,
        grid=grid,
        in_specs=[
            pl.BlockSpec((ROW_BLK, D), lambda i: (i, 0)),
            pl.BlockSpec((D, D), lambda i: (0, 0)),
            pl.BlockSpec((1, D), lambda i: (0, 0)),
        ],
        out_specs=pl.BlockSpec((ROW_BLK, D // 2), lambda i: (i, 0)),
        out_shape=jax.ShapeDtypeStruct((V, D // 2), jnp.int32),
    )(table, W, b.reshape(1, D))


def _gather_rows(t2p, ids_flat, D):
    info = plsc.get_sparse_core_info()
    NC, NS = info.num_cores, info.num_subcores
    NW = NC * NS
    N = ids_flat.shape[0]
    DW = t2p.shape[1]
    assert N % (NW * CH) == 0 and D % (2 * LANES) == 0
    b_per_w = N // NW
    n_ch = b_per_w // CH
    n_grp = D // (2 * LANES)
    mesh = plsc.VectorSubcoreMesh(core_axis_name="c", subcore_axis_name="s")

    assert n_ch % 2 == 0 and n_ch >= 4

    @functools.partial(
        pl.kernel,
        mesh=mesh,
        compiler_params=pltpu.CompilerParams(needs_layout_passes=False),
        out_type=jax.ShapeDtypeStruct((N, D), jnp.float32),
        scratch_types=[
            pltpu.VMEM((b_per_w,), jnp.int32),
            [pltpu.VMEM((CH, DW), jnp.int32) for _ in range(2)],
            [pltpu.VMEM((CH, D), jnp.float32) for _ in range(2)],
            [pltpu.SemaphoreType.DMA for _ in range(2)],
            [pltpu.SemaphoreType.DMA for _ in range(2)],
        ],
    )
    def k(t2_hbm, idx_hbm, out_hbm, idx_v, gbufs, obufs, gsems, ssems):
        wid = lax.axis_index("s") * NC + lax.axis_index("c")
        base = wid * b_per_w
        pltpu.sync_copy(idx_hbm.at[pl.ds(base, b_per_w)], idx_v)

        def start_gather(c, b):
            pltpu.async_copy(
                t2_hbm.at[idx_v.at[pl.ds(c * CH, CH)]], gbufs[b], gsems[b])

        def wait_gather(b):
            pltpu.make_async_copy(t2_hbm.at[pl.ds(0, CH)], gbufs[b],
                                  gsems[b]).wait()

        def start_scatter(c, b):
            pltpu.async_copy(obufs[b], out_hbm.at[pl.ds(base + c * CH, CH)],
                             ssems[b])

        def wait_scatter(b):
            pltpu.make_async_copy(obufs[b], out_hbm.at[pl.ds(0, CH)],
                                  ssems[b]).wait()

        def convert(gb, ob):
            mask = jnp.int32(-65536)

            def row_body(r, carry):
                for g in range(n_grp):
                    x = gb[r, pl.ds(LANES * g, LANES)]
                    ob[r, pl.ds(2 * LANES * g, LANES)] = plsc.bitcast(
                        x << 16, jnp.float32)
                    ob[r, pl.ds(2 * LANES * g + LANES, LANES)] = plsc.bitcast(
                        x & mask, jnp.float32)
                return carry

            lax.fori_loop(0, CH, row_body, 0)

        start_gather(0, 0)
        start_gather(1, 1)

        def body(p, carry):
            for b in range(2):
                c = 2 * p + b
                wait_gather(b)

                @pl.when(c >= 2)
                def _():
                    wait_scatter(b)

                convert(gbufs[b], obufs[b])
                start_scatter(c, b)

                @pl.when(c + 2 < n_ch)
                def _():
                    start_gather(c + 2, b)
            return carry

        lax.fori_loop(0, n_ch // 2, body, 0)
        for b in range(2):
            wait_scatter(b)

    return k(t2p, ids_flat)


def kernel(input_ids, table, W, b):
    B, L = input_ids.shape
    D = table.shape[1]
    perm = _column_perm(D)
    t2p = _transform_table(table, W[perm, :], b[perm])
    ids_flat = input_ids.reshape(B * L).astype(jnp.int32)
    out_flat = _gather_rows(t2p, ids_flat, D)
    return out_flat.reshape(B, L, D)


# speedup vs baseline: 1.4970x; 1.4970x over previous
import functools

import jax
import jax.numpy as jnp
from jax import lax
from jax.experimental import pallas as pl
from jax.experimental.pallas import tpu as pltpu
from jax.experimental.pallas import tpu_sc as plsc

ROW_BLK = 512
CH = 32


def _mm_kernel(t_ref, w_ref, b_ref, o_ref):
    o_ref[...] = lax.dot_general(
        t_ref[...].astype(jnp.bfloat16), w_ref[...].astype(jnp.bfloat16),
        (((1,), (1,)), ((), ())),
        preferred_element_type=jnp.float32) + b_ref[...]


def _transform_table(table, W, b):
    V, D = table.shape
    grid = (pl.cdiv(V, ROW_BLK),)
    return pl.pallas_call(
        _mm_kernel,
        grid=grid,
        in_specs=[
            pl.BlockSpec((ROW_BLK, D), lambda i: (i, 0)),
            pl.BlockSpec((D, D), lambda i: (0, 0)),
            pl.BlockSpec((1, D), lambda i: (0, 0)),
        ],
        out_specs=pl.BlockSpec((ROW_BLK, D), lambda i: (i, 0)),
        out_shape=jax.ShapeDtypeStruct((V, D), jnp.float32),
    )(table, W, b.reshape(1, D))


def _gather_rows(t2, ids_flat):
    info = plsc.get_sparse_core_info()
    NC, NS = info.num_cores, info.num_subcores
    NW = NC * NS
    N = ids_flat.shape[0]
    D = t2.shape[1]
    assert N % (NW * CH) == 0
    b_per_w = N // NW
    n_ch = b_per_w // CH
    mesh = plsc.VectorSubcoreMesh(core_axis_name="c", subcore_axis_name="s")

    NBUF = 4
    assert n_ch % NBUF == 0 and n_ch >= 2 * NBUF

    @functools.partial(
        pl.kernel,
        mesh=mesh,
        out_type=jax.ShapeDtypeStruct((N, D), jnp.float32),
        scratch_types=[
            pltpu.VMEM((b_per_w,), jnp.int32),
            [pltpu.VMEM((CH, D), jnp.float32) for _ in range(NBUF)],
            [pltpu.SemaphoreType.DMA for _ in range(NBUF)],
            [pltpu.SemaphoreType.DMA for _ in range(NBUF)],
        ],
    )
    def k(t2_hbm, idx_hbm, out_hbm, idx_v, bufs, gsems, ssems):
        wid = lax.axis_index("s") * NC + lax.axis_index("c")
        base = wid * b_per_w
        pltpu.sync_copy(idx_hbm.at[pl.ds(base, b_per_w)], idx_v)

        def start_gather(c, b):
            pltpu.async_copy(
                t2_hbm.at[idx_v.at[pl.ds(c * CH, CH)]], bufs[b], gsems[b])

        def wait_gather(b):
            pltpu.make_async_copy(t2_hbm.at[pl.ds(0, CH)], bufs[b],
                                  gsems[b]).wait()

        def start_scatter(c, b):
            pltpu.async_copy(bufs[b], out_hbm.at[pl.ds(base + c * CH, CH)],
                             ssems[b])

        def wait_scatter(b):
            pltpu.make_async_copy(bufs[b], out_hbm.at[pl.ds(0, CH)],
                                  ssems[b]).wait()

        start_gather(0, 0)
        start_gather(1, 1)

        def body(p, carry):
            for b in range(NBUF):
                c = NBUF * p + b
                wait_gather(b)
                start_scatter(c, b)
                g = c + 2
                b2 = (b + 2) % NBUF

                @pl.when(g < n_ch)
                def _():
                    @pl.when(c >= 2)
                    def _():
                        wait_scatter(b2)
                    start_gather(g, b2)
            return carry

        lax.fori_loop(0, n_ch // NBUF, body, 0)
        for b in range(NBUF):
            wait_scatter(b)

    return k(t2, ids_flat)


def kernel(input_ids, table, W, b):
    B, L = input_ids.shape
    t2 = _transform_table(table, W, b)
    ids_flat = input_ids.reshape(B * L).astype(jnp.int32)
    out_flat = _gather_rows(t2, ids_flat)
    return out_flat.reshape(B, L, -1)


# speedup vs baseline: 1.5022x; 1.0035x over previous
import functools

import jax
import jax.numpy as jnp
from jax import lax
from jax.experimental import pallas as pl
from jax.experimental.pallas import tpu as pltpu
from jax.experimental.pallas import tpu_sc as plsc

ROW_BLK = 512
CH = 32


def _mm_kernel(t_ref, w_ref, b_ref, o_ref):
    o_ref[...] = lax.dot_general(
        t_ref[...].astype(jnp.bfloat16), w_ref[...].astype(jnp.bfloat16),
        (((1,), (1,)), ((), ())),
        preferred_element_type=jnp.float32) + b_ref[...]


def _transform_table(table, W, b):
    V, D = table.shape
    grid = (pl.cdiv(V, ROW_BLK),)
    return pl.pallas_call(
        _mm_kernel,
        grid=grid,
        in_specs=[
            pl.BlockSpec((ROW_BLK, D), lambda i: (i, 0)),
            pl.BlockSpec((D, D), lambda i: (0, 0)),
            pl.BlockSpec((1, D), lambda i: (0, 0)),
        ],
        out_specs=pl.BlockSpec((ROW_BLK, D), lambda i: (i, 0)),
        out_shape=jax.ShapeDtypeStruct((V, D), jnp.float32),
    )(table, W, b.reshape(1, D))


def _gather_rows(t2, ids_flat):
    info = plsc.get_sparse_core_info()
    NC, NS = info.num_cores, info.num_subcores
    NW = NC * NS
    N = ids_flat.shape[0]
    D = t2.shape[1]
    assert N % (NW * CH) == 0
    b_per_w = N // NW
    n_ch = b_per_w // CH
    mesh = plsc.VectorSubcoreMesh(core_axis_name="c", subcore_axis_name="s")

    NBUF = 4
    LOOK = 3
    assert n_ch % NBUF == 0 and n_ch >= 2 * NBUF and 1 <= LOOK < NBUF

    @functools.partial(
        pl.kernel,
        mesh=mesh,
        out_type=jax.ShapeDtypeStruct((N, D), jnp.float32),
        scratch_types=[
            pltpu.VMEM((b_per_w,), jnp.int32),
            [pltpu.VMEM((CH, D), jnp.float32) for _ in range(NBUF)],
            [pltpu.SemaphoreType.DMA for _ in range(NBUF)],
            [pltpu.SemaphoreType.DMA for _ in range(NBUF)],
        ],
    )
    def k(t2_hbm, idx_hbm, out_hbm, idx_v, bufs, gsems, ssems):
        wid = lax.axis_index("s") * NC + lax.axis_index("c")
        base = wid * b_per_w
        pltpu.sync_copy(idx_hbm.at[pl.ds(base, b_per_w)], idx_v)

        def start_gather(c, b):
            pltpu.async_copy(
                t2_hbm.at[idx_v.at[pl.ds(c * CH, CH)]], bufs[b], gsems[b])

        def wait_gather(b):
            pltpu.make_async_copy(t2_hbm.at[pl.ds(0, CH)], bufs[b],
                                  gsems[b]).wait()

        def start_scatter(c, b):
            pltpu.async_copy(bufs[b], out_hbm.at[pl.ds(base + c * CH, CH)],
                             ssems[b])

        def wait_scatter(b):
            pltpu.make_async_copy(bufs[b], out_hbm.at[pl.ds(0, CH)],
                                  ssems[b]).wait()

        for c0 in range(LOOK):
            start_gather(c0, c0 % NBUF)

        def body(p, carry):
            for j in range(NBUF):
                c = NBUF * p + j
                wait_gather(j)
                start_scatter(c, j)
                g = c + LOOK
                bg = (j + LOOK) % NBUF

                @pl.when(g < n_ch)
                def _():
                    @pl.when(g >= NBUF)
                    def _():
                        wait_scatter(bg)
                    start_gather(g, bg)
            return carry

        lax.fori_loop(0, n_ch // NBUF, body, 0)
        for b in range(NBUF):
            wait_scatter(b)

    return k(t2, ids_flat)


def kernel(input_ids, table, W, b):
    B, L = input_ids.shape
    t2 = _transform_table(table, W, b)
    ids_flat = input_ids.reshape(B * L).astype(jnp.int32)
    out_flat = _gather_rows(t2, ids_flat)
    return out_flat.reshape(B, L, -1)
